# baseline (speedup 1.0000x reference)
import jax
import jax.numpy as jnp
from jax import lax
from jax.experimental import pallas as pl
from jax.experimental.pallas import tpu as pltpu
from jax.experimental.pallas import tpu_sc as plsc

_TEST = 0.0

_N = 32768
_D = 256
_NC = 2
_NS = 16
_NW = _NC * _NS
_RPW = _N // _NW
_CH = 16
_NBUF = 8
_GROUPS = _RPW // (_CH * _NBUF)
_UNROLL = 4


def _fix_one_row(c_v, d_v, r, lane0):
    tc = c_v[r, pl.ds(0, 16)]
    td = d_v[r, pl.ds(0, 16)]
    lo = tc - td
    hi = tc + td
    left = lo <= _TEST
    right = hi > _TEST
    l_hi = jnp.minimum(hi, _TEST)
    l_c = (lo + l_hi) * 0.5
    l_d = (l_hi - lo) * 0.5
    r_lo = jnp.maximum(lo, _TEST)
    r_c = (r_lo + hi) * 0.5
    r_d = (hi - r_lo) * 0.5
    lo_l = l_c - l_d
    hi_l = l_c + l_d
    lo_r = r_c - r_d
    hi_r = r_c + r_d
    both = left & right
    new_lo = jnp.where(both, jnp.minimum(lo_l, lo_r),
                       jnp.where(left, lo_l, lo_r))
    new_hi = jnp.where(both, jnp.maximum(hi_l, hi_r),
                       jnp.where(left, hi_l, hi_r))
    c_v[r, pl.ds(0, 16)] = jnp.where(lane0, (new_lo + new_hi) * 0.5, tc)
    d_v[r, pl.ds(0, 16)] = jnp.where(lane0, (new_hi - new_lo) * 0.5, td)


def _fix_rows(c_v, d_v):
    lane0 = lax.iota(jnp.int32, 16) == 0

    def block(i, _):
        for u in range(_UNROLL):
            _fix_one_row(c_v, d_v, i * _UNROLL + u, lane0)
        return 0

    lax.fori_loop(0, _CH // _UNROLL, block, 0)


def _sc_body(c_hbm, d_hbm, out_hbm, bufs, insems, outsems):
    wid = lax.axis_index("s") * _NC + lax.axis_index("c")
    base = wid * _RPW

    def in_start(chunk, b):
        rows = pl.ds(base + chunk * _CH, _CH)
        c_v, d_v = bufs[b]
        pltpu.async_copy(c_hbm.at[rows, :], c_v, insems[b])
        pltpu.async_copy(d_hbm.at[rows, :], d_v, insems[b])

    def in_wait(chunk, b):
        rows = pl.ds(base + chunk * _CH, _CH)
        c_v, d_v = bufs[b]
        pltpu.make_async_copy(c_hbm.at[rows, :], c_v, insems[b]).wait()
        pltpu.make_async_copy(d_hbm.at[rows, :], d_v, insems[b]).wait()

    def out_start(chunk, b):
        rows = pl.ds(base + chunk * _CH, _CH)
        c_v, d_v = bufs[b]
        pltpu.async_copy(c_v, out_hbm.at[0, rows, :], outsems[b])
        pltpu.async_copy(d_v, out_hbm.at[1, rows, :], outsems[b])

    def out_wait(chunk, b):
        rows = pl.ds(base + chunk * _CH, _CH)
        c_v, d_v = bufs[b]
        pltpu.make_async_copy(c_v, out_hbm.at[0, rows, :], outsems[b]).wait()
        pltpu.make_async_copy(d_v, out_hbm.at[1, rows, :], outsems[b]).wait()

    for b in range(_NBUF):
        in_start(b, b)

    def group(g, _):
        for b in range(_NBUF):
            chunk = g * _NBUF + b
            in_wait(chunk, b)
            c_v, d_v = bufs[b]
            _fix_rows(c_v, d_v)
            out_start(chunk, b)
        for b in range(_NBUF):
            @pl.when(g + 1 < _GROUPS)
            def _():
                out_wait(g * _NBUF + b, b)
                in_start((g + 1) * _NBUF + b, b)
        return 0

    lax.fori_loop(0, _GROUPS, group, 0)

    for b in range(_NBUF):
        out_wait((_GROUPS - 1) * _NBUF + b, b)


def kernel(c, delta):
    mesh = plsc.VectorSubcoreMesh(core_axis_name="c", subcore_axis_name="s")
    bufs = [
        (pltpu.VMEM((_CH, _D), jnp.float32), pltpu.VMEM((_CH, _D), jnp.float32))
        for _ in range(_NBUF)
    ]
    f = pl.kernel(
        _sc_body,
        out_type=jax.ShapeDtypeStruct((2, _N, _D), jnp.float32),
        mesh=mesh,
        scratch_types=[
            bufs,
            [pltpu.SemaphoreType.DMA for _ in range(_NBUF)],
            [pltpu.SemaphoreType.DMA for _ in range(_NBUF)],
        ],
    )
    return f(c, delta)


# speedup vs baseline: 1.0186x; 1.0186x over previous
import jax
import jax.numpy as jnp
from jax import lax
from jax.experimental import pallas as pl
from jax.experimental.pallas import tpu as pltpu
from jax.experimental.pallas import tpu_sc as plsc

_TEST = 0.0

_N = 32768
_D = 256
_NC = 2
_NS = 16
_NW = _NC * _NS
_RPW = _N // _NW
_CH = 32
_NBUF = 4
_GROUPS = _RPW // (_CH * _NBUF)
_LOOK = 2
_UNROLL = 4


def _fix_one_row(c_v, d_v, r, lane0):
    tc = c_v[r, pl.ds(0, 16)]
    td = d_v[r, pl.ds(0, 16)]
    lo = tc - td
    hi = tc + td
    left = lo <= _TEST
    right = hi > _TEST
    l_hi = jnp.minimum(hi, _TEST)
    l_c = (lo + l_hi) * 0.5
    l_d = (l_hi - lo) * 0.5
    r_lo = jnp.maximum(lo, _TEST)
    r_c = (r_lo + hi) * 0.5
    r_d = (hi - r_lo) * 0.5
    lo_l = l_c - l_d
    hi_l = l_c + l_d
    lo_r = r_c - r_d
    hi_r = r_c + r_d
    both = left & right
    new_lo = jnp.where(both, jnp.minimum(lo_l, lo_r),
                       jnp.where(left, lo_l, lo_r))
    new_hi = jnp.where(both, jnp.maximum(hi_l, hi_r),
                       jnp.where(left, hi_l, hi_r))
    c_v[r, pl.ds(0, 16)] = jnp.where(lane0, (new_lo + new_hi) * 0.5, tc)
    d_v[r, pl.ds(0, 16)] = jnp.where(lane0, (new_hi - new_lo) * 0.5, td)


def _fix_rows(c_v, d_v):
    lane0 = lax.iota(jnp.int32, 16) == 0

    def block(i, _):
        for u in range(_UNROLL):
            _fix_one_row(c_v, d_v, i * _UNROLL + u, lane0)
        return 0

    lax.fori_loop(0, _CH // _UNROLL, block, 0)


def _sc_body(c_hbm, d_hbm, out_hbm, bufs, insems, outsems):
    wid = lax.axis_index("s") * _NC + lax.axis_index("c")
    base = wid * _RPW

    def in_start(chunk, b):
        rows = pl.ds(base + chunk * _CH, _CH)
        c_v, d_v = bufs[b]
        pltpu.async_copy(c_hbm.at[rows, :], c_v, insems[b])
        pltpu.async_copy(d_hbm.at[rows, :], d_v, insems[b])

    def in_wait(chunk, b):
        rows = pl.ds(base + chunk * _CH, _CH)
        c_v, d_v = bufs[b]
        pltpu.make_async_copy(c_hbm.at[rows, :], c_v, insems[b]).wait()
        pltpu.make_async_copy(d_hbm.at[rows, :], d_v, insems[b]).wait()

    def out_start(chunk, b):
        rows = pl.ds(base + chunk * _CH, _CH)
        c_v, d_v = bufs[b]
        pltpu.async_copy(c_v, out_hbm.at[0, rows, :], outsems[b])
        pltpu.async_copy(d_v, out_hbm.at[1, rows, :], outsems[b])

    def out_wait(chunk, b):
        rows = pl.ds(base + chunk * _CH, _CH)
        c_v, d_v = bufs[b]
        pltpu.make_async_copy(c_v, out_hbm.at[0, rows, :], outsems[b]).wait()
        pltpu.make_async_copy(d_v, out_hbm.at[1, rows, :], outsems[b]).wait()

    chunks = _RPW // _CH
    for j in range(_LOOK):
        in_start(j, j)

    def group(g, _):
        for b in range(_NBUF):
            k = g * _NBUF + b
            in_wait(k, b)
            c_v, d_v = bufs[b]
            _fix_rows(c_v, d_v)
            out_start(k, b)
            wb = (b - _LOOK) % _NBUF

            @pl.when(k - _LOOK >= 0)
            def _():
                out_wait(k - _LOOK, wb)
            nb = (b + _LOOK) % _NBUF

            @pl.when(k + _LOOK < chunks)
            def _():
                in_start(k + _LOOK, nb)
        return 0

    lax.fori_loop(0, _GROUPS, group, 0)

    for k in range(chunks - _LOOK, chunks):
        out_wait(k, k % _NBUF)


def kernel(c, delta):
    mesh = plsc.VectorSubcoreMesh(core_axis_name="c", subcore_axis_name="s")
    bufs = [
        (pltpu.VMEM((_CH, _D), jnp.float32), pltpu.VMEM((_CH, _D), jnp.float32))
        for _ in range(_NBUF)
    ]
    f = pl.kernel(
        _sc_body,
        out_type=jax.ShapeDtypeStruct((2, _N, _D), jnp.float32),
        mesh=mesh,
        scratch_types=[
            bufs,
            [pltpu.SemaphoreType.DMA for _ in range(_NBUF)],
            [pltpu.SemaphoreType.DMA for _ in range(_NBUF)],
        ],
    )
    return f(c, delta)
